# Initial kernel scaffold; baseline (speedup 1.0000x reference)
import jax
import jax.numpy as jnp
from jax.experimental import pallas as pl


def kernel(ent_seed_sr, ent_seed_tg, attribute_triples_sr, attribute_triples_tg, edges_sr, edges_tg, ev_edges_sr, vv_edges_sr, ev_edges_tg, vv_edges_tg, val_feats, att_feats, ent_feats_sr, ent_feats_tg, W_ve, v_gcn1_W, v_gcn1_b, v_gcn2_W, v_gcn2_b, gat1_W, gat1_as, gat1_ad, gat1_b, gatr_W, gatr_as, gatr_ad, gatr_b, e_gcn1_W, e_gcn1_b, e_gcn2_W, e_gcn2_b):
    raise NotImplementedError("write your pallas kernel here")


# speedup vs baseline: 1.0000x; 1.0000x over previous
import jax
import jax.numpy as jnp
from jax.experimental import pallas as pl


def _gcn_conv(x, edge_index, W, b, n):
    src = jnp.concatenate([edge_index[0], jnp.arange(n)])
    dst = jnp.concatenate([edge_index[1], jnp.arange(n)])
    h = x @ W
    deg = jax.ops.segment_sum(jnp.ones_like(dst, dtype=x.dtype), dst, num_segments=n)
    dinv = jnp.where(deg > 0, jax.lax.rsqrt(deg), 0.0)
    norm = dinv[src] * dinv[dst]
    out = jax.ops.segment_sum(h[src] * norm[:, None], dst, num_segments=n)
    return out + b


def _gat_conv(x, edge_index, W, a_src, a_dst, b, n):
    src = jnp.concatenate([edge_index[0], jnp.arange(n)])
    dst = jnp.concatenate([edge_index[1], jnp.arange(n)])
    h = x @ W
    e = jax.nn.leaky_relu((h @ a_src)[src] + (h @ a_dst)[dst], 0.2)
    emax = jax.lax.stop_gradient(jax.ops.segment_max(e, dst, num_segments=n))
    emax = jnp.where(jnp.isfinite(emax), emax, 0.0)
    ee = jnp.exp(e - emax[dst])
    den = jax.ops.segment_sum(ee, dst, num_segments=n)
    alpha = ee / (den[dst] + 1e-16)
    out = jax.ops.segment_sum(h[src] * alpha[:, None], dst, num_segments=n)
    return out + b


def _l2norm(x):
    return x / jnp.clip(jnp.linalg.norm(x, axis=-1, keepdims=True), 1e-12)


def _copy_kernel(x_ref, o_ref):
    o_ref[...] = x_ref[...]


def _pl_copy(x):
    return pl.pallas_call(
        _copy_kernel, out_shape=jax.ShapeDtypeStruct(x.shape, x.dtype))(x)


def kernel(ent_seed_sr, ent_seed_tg, attribute_triples_sr, attribute_triples_tg, edges_sr, edges_tg, ev_edges_sr, vv_edges_sr, ev_edges_tg, vv_edges_tg, val_feats, att_feats, ent_feats_sr, ent_feats_tg, W_ve, v_gcn1_W, v_gcn1_b, v_gcn2_W, v_gcn2_b, gat1_W, gat1_as, gat1_ad, gat1_b, gatr_W, gatr_as, gatr_ad, gatr_b, e_gcn1_W, e_gcn1_b, e_gcn2_W, e_gcn2_b):
    def value_encoder(triples, ent_edges, val_edges, ent_feats):
        val = triples[:, 1]
        att = triples[:, 2]
        vf = jnp.concatenate([att_feats[att], val_feats[val]], axis=1) @ W_ve
        nodes = jnp.concatenate([ent_feats, vf], axis=0)
        n = nodes.shape[0]
        vei = val_edges.T
        nodes = _gcn_conv(nodes, vei, v_gcn1_W, v_gcn1_b, n)
        nodes = _gcn_conv(nodes, vei, v_gcn2_W, v_gcn2_b, n)
        eei = ent_edges.T
        nodes = (_gat_conv(nodes, eei, gat1_W, gat1_as, gat1_ad, gat1_b, n)
                 + _gat_conv(nodes, eei, gatr_W, gatr_as, gatr_ad, gatr_b, n))
        num_ent = ent_feats.shape[0]
        return nodes[:num_ent] + ent_feats

    def entity_encoder(x, edges):
        ei = edges.T
        n = x.shape[0]
        h = _gcn_conv(x, ei, e_gcn1_W, e_gcn1_b, n) + x
        h = _gcn_conv(h, ei, e_gcn2_W, e_gcn2_b, n) + x
        return h

    efs = value_encoder(attribute_triples_sr, ev_edges_sr, vv_edges_sr, ent_feats_sr)
    eft = value_encoder(attribute_triples_tg, ev_edges_tg, vv_edges_tg, ent_feats_tg)
    efs = entity_encoder(efs, edges_sr)
    eft = entity_encoder(eft, edges_tg)
    efs = _l2norm(_pl_copy(efs))
    eft = _l2norm(_pl_copy(eft))
    return (efs[ent_seed_sr], eft[ent_seed_tg], efs, eft)


# speedup vs baseline: 4.7136x; 4.7134x over previous
import functools

import jax
import jax.numpy as jnp
from jax import lax
from jax.experimental import pallas as pl
from jax.experimental.pallas import tpu as pltpu
from jax.experimental.pallas import tpu_sc as plsc

NC, NS, L = 2, 16, 16
NW = NC * NS
D = 128
R = 128


def _mesh():
    return plsc.VectorSubcoreMesh(core_axis_name="c", subcore_axis_name="s")


def _wid():
    return lax.axis_index("s") * NC + lax.axis_index("c")


def _ceil(a, b):
    return (a + b - 1) // b


def _rup(a, b):
    return _ceil(a, b) * b


@functools.lru_cache(maxsize=None)
def _make_gather(n_idx, na, nb):
    two = nb is not None
    nb_full = n_idx // R
    tail = n_idx - nb_full * R
    T = _ceil(nb_full, NW)

    scratch = [pltpu.VMEM((R,), jnp.int32), pltpu.VMEM((R, D), jnp.float32)]
    if two:
        scratch += [pltpu.VMEM((R,), jnp.int32), pltpu.VMEM((R, D), jnp.float32)]
    scratch += [pltpu.SemaphoreType.DMA]

    def body(*refs):
        if two:
            a_hbm, ia_hbm, b_hbm, ib_hbm, out_hbm, ia_v, rows_a, ib_v, rows_b, sem = refs
        else:
            a_hbm, ia_hbm, out_hbm, ia_v, rows_a, sem = refs
        w = _wid()

        def zfill(ref, nrows):
            if nrows < R:
                for o in range((nrows // L) * L, R, L):
                    ref[pl.ds(o, L)] = jnp.zeros((L,), jnp.int32)

        def process(start, nrows):
            zfill(ia_v, nrows)
            pltpu.sync_copy(ia_hbm.at[pl.ds(start, nrows)], ia_v.at[pl.ds(0, nrows)])
            pltpu.async_copy(a_hbm.at[ia_v], rows_a, sem).wait()
            if two:
                zfill(ib_v, nrows)
                pltpu.sync_copy(ib_hbm.at[pl.ds(start, nrows)], ib_v.at[pl.ds(0, nrows)])
                pltpu.async_copy(b_hbm.at[ib_v], rows_b, sem).wait()

                def add_row(r, _):
                    for j in range(D // L):
                        s = pl.ds(j * L, L)
                        rows_a[r, s] = rows_a[r, s] + rows_b[r, s]
                    return 0

                lax.fori_loop(0, nrows, add_row, 0)
            pltpu.sync_copy(rows_a.at[pl.ds(0, nrows)], out_hbm.at[pl.ds(start, nrows)])

        for t in range(T):
            b = w + NW * t
            if (t + 1) * NW <= nb_full:
                process(b * R, R)
            else:
                @pl.when(b < nb_full)
                def _():
                    process(b * R, R)
        if tail:
            @pl.when(w == NW - 1)
            def _():
                process(nb_full * R, tail)

    return pl.kernel(
        body,
        out_type=jax.ShapeDtypeStruct((n_idx, D), jnp.float32),
        mesh=_mesh(),
        scratch_types=scratch,
        compiler_params=pltpu.CompilerParams(needs_layout_passes=False),
    )


def _sc_gather(a, ia):
    return _make_gather(ia.shape[0], a.shape[0], None)(a, ia)


def _sc_gather2(a, ia, b, ib):
    return _make_gather(ia.shape[0], a.shape[0], b.shape[0])(a, ia, b, ib)


@functools.lru_cache(maxsize=None)
def _make_seghist(E, N, with_vals):
    NA = _rup(N + 1, 2048)
    stripe = NA // NS
    nb_full = E // R
    tail = E - nb_full * R
    T = _ceil(nb_full, NW)

    scratch = [
        pltpu.VMEM((R,), jnp.int32),
        pltpu.VMEM((R,), jnp.float32),
        pltpu.VMEM((R,), jnp.float32),
        pltpu.VMEM_SHARED((NA,), jnp.float32),
        pltpu.SemaphoreType.DMA,
    ]

    def body(*refs):
        if with_vals:
            dst_hbm, vals_hbm, out_hbm, ldst_v, vals_v, zb, acc, sem = refs
        else:
            dst_hbm, out_hbm, ldst_v, vals_v, zb, acc, sem = refs
        cid = lax.axis_index("c")
        sid = lax.axis_index("s")
        w = sid * NC + cid

        for j in range(R // L):
            zb[pl.ds(j * L, L)] = jnp.zeros((L,), jnp.float32)
        if not with_vals:
            for j in range(R // L):
                vals_v[pl.ds(j * L, L)] = jnp.ones((L,), jnp.float32)

        def zrow(t, _):
            pltpu.sync_copy(zb, acc.at[pl.ds(sid * stripe + t * R, R)])
            return 0
        lax.fori_loop(0, stripe // R, zrow, 0)
        plsc.subcore_barrier()

        def process(start, nrows):
            if nrows < R:
                for o in range((nrows // L) * L, R, L):
                    ldst_v[pl.ds(o, L)] = jnp.full((L,), N, jnp.int32)
            pltpu.sync_copy(dst_hbm.at[pl.ds(start, nrows)], ldst_v.at[pl.ds(0, nrows)])
            if with_vals:
                pltpu.sync_copy(vals_hbm.at[pl.ds(start, nrows)], vals_v.at[pl.ds(0, nrows)])
            pltpu.sync_copy(vals_v, acc.at[ldst_v], add=True)

        for t in range(T):
            b = w + NW * t
            if (t + 1) * NW <= nb_full:
                process(b * R, R)
            else:
                @pl.when(b < nb_full)
                def _():
                    process(b * R, R)
        if tail:
            @pl.when(w == NW - 1)
            def _():
                process(nb_full * R, tail)

        plsc.subcore_barrier()
        def wrow(t, _):
            o = sid * stripe + t * R
            pltpu.sync_copy(acc.at[pl.ds(o, R)], zb)
            pltpu.sync_copy(zb, out_hbm.at[cid, pl.ds(o, R)])
            return 0
        lax.fori_loop(0, stripe // R, wrow, 0)

    return pl.kernel(
        body,
        out_type=jax.ShapeDtypeStruct((2, NA), jnp.float32),
        mesh=_mesh(),
        scratch_types=scratch,
        compiler_params=pltpu.CompilerParams(needs_layout_passes=False),
    ), NA


def _sc_seghist(dst, n, vals=None):
    kf, na = _make_seghist(dst.shape[0], n, vals is not None)
    if vals is None:
        return kf(dst), na
    return kf(dst, vals), na


@functools.lru_cache(maxsize=None)
def _make_edge_pass(E, N, phase):
    nb_full = E // R
    tail = E - nb_full * R
    T = _ceil(nb_full, NW)

    scratch = [
        pltpu.VMEM((N,), jnp.float32),
        pltpu.VMEM((R,), jnp.int32),
        pltpu.VMEM((R,), jnp.float32),
        pltpu.SemaphoreType.DMA,
    ]
    if phase == 2:
        scratch.insert(2, pltpu.VMEM((R,), jnp.float32))
        scratch.insert(3, pltpu.VMEM((16,), jnp.float32))

    def body(*refs):
        if phase == 1:
            tab_hbm, idx_hbm, out_hbm, tab_v, idx_v, out_v, sem = refs
        else:
            tab_hbm, idx_hbm, p1_hbm, m_hbm, out_hbm, tab_v, idx_v, p1_v, m_v, out_v, sem = refs
        w = _wid()
        pltpu.sync_copy(tab_hbm, tab_v)
        if phase == 2:
            pltpu.sync_copy(m_hbm.at[pl.ds(0, L)], m_v)

        def process(start, nrows):
            if nrows < R:
                for o in range((nrows // L) * L, R, L):
                    idx_v[pl.ds(o, L)] = jnp.zeros((L,), jnp.int32)
            pltpu.sync_copy(idx_hbm.at[pl.ds(start, nrows)], idx_v.at[pl.ds(0, nrows)])
            if phase == 2:
                pltpu.sync_copy(p1_hbm.at[pl.ds(start, nrows)], p1_v.at[pl.ds(0, nrows)])
                mv = m_v[pl.ds(0, L)]
            for j in range(R // L):
                s = pl.ds(j * L, L)
                g = plsc.load_gather(tab_v, [idx_v[s]])
                if phase == 1:
                    out_v[s] = g
                else:
                    x = p1_v[s] + g
                    x = jnp.where(x >= 0.0, x, 0.2 * x)
                    out_v[s] = jnp.exp(x - mv)
            pltpu.sync_copy(out_v.at[pl.ds(0, nrows)], out_hbm.at[pl.ds(start, nrows)])

        for t in range(T):
            b = w + NW * t
            if (t + 1) * NW <= nb_full:
                process(b * R, R)
            else:
                @pl.when(b < nb_full)
                def _():
                    process(b * R, R)
        if tail:
            @pl.when(w == NW - 1)
            def _():
                process(nb_full * R, tail)

    return pl.kernel(
        body,
        out_type=jax.ShapeDtypeStruct((E,), jnp.float32),
        mesh=_mesh(),
        scratch_types=scratch,
        compiler_params=pltpu.CompilerParams(needs_layout_passes=False),
    )


C_CHUNK = 4080
CA = C_CHUNK + 16


@functools.lru_cache(maxsize=None)
def _make_rowsum(E, N, Ns, with_scale):
    assert E % 2048 == 0
    E16 = E // NS
    nchunks = _ceil(N, C_CHUNK)
    nv = E16 // L

    scratch = [
        pltpu.VMEM((E16 + R + L,), jnp.int32),
        pltpu.VMEM((E16 + R + L,), jnp.int32),
        pltpu.VMEM((R, D), jnp.float32),
        pltpu.VMEM((R, D), jnp.float32),
        pltpu.VMEM((1, R), jnp.int32),
        pltpu.VMEM_SHARED((CA, D), jnp.float32),
        pltpu.SemaphoreType.DMA,
    ]
    if with_scale:
        scratch.insert(2, pltpu.VMEM((E16 + R + L,), jnp.float32))

    def body(*refs):
        if with_scale:
            (h_hbm, src_hbm, dst_hbm, scale_hbm, out_hbm,
             src_v, dst_v, scale_v, rows_v, zb, ldst2, acc, sem) = refs
        else:
            (h_hbm, src_hbm, dst_hbm, out_hbm,
             src_v, dst_v, rows_v, zb, ldst2, acc, sem) = refs
        cid = lax.axis_index("c")
        sid = lax.axis_index("s")

        def zrow(r, _):
            for j in range(D // L):
                zb[r, pl.ds(j * L, L)] = jnp.zeros((L,), jnp.float32)
            return 0
        lax.fori_loop(0, R, zrow, 0)

        pltpu.sync_copy(src_hbm.at[pl.ds(sid * E16, E16)], src_v.at[pl.ds(0, E16)])
        pltpu.sync_copy(dst_hbm.at[pl.ds(sid * E16, E16)], dst_v.at[pl.ds(0, E16)])
        if with_scale:
            pltpu.sync_copy(scale_hbm.at[pl.ds(sid * E16, E16)], scale_v.at[pl.ds(0, E16)])

        for q in range(nchunks):
            base = q * C_CHUNK
            valid = min(C_CHUNK, N - base)

            @pl.when(cid == (q % 2))
            def _():
                for t in range(CA // NS // R):
                    pltpu.sync_copy(zb, acc.at[pl.ds(sid * (CA // NS) + t * R, R)])
                plsc.subcore_barrier()

                if q == 0:
                    pass
                else:
                    pltpu.sync_copy(src_hbm.at[pl.ds(sid * E16, E16)],
                                    src_v.at[pl.ds(0, E16)])
                    pltpu.sync_copy(dst_hbm.at[pl.ds(sid * E16, E16)],
                                    dst_v.at[pl.ds(0, E16)])
                    if with_scale:
                        pltpu.sync_copy(scale_hbm.at[pl.ds(sid * E16, E16)],
                                        scale_v.at[pl.ds(0, E16)])

                trash = E16 + R

                def compact(t, cnt):
                    s = pl.ds(t * L, L)
                    dv = dst_v[s]
                    sv = src_v[s]
                    msk = (dv >= base) & (dv < base + C_CHUNK)
                    mi = msk.astype(jnp.int32)
                    pos = jnp.where(msk, cnt + plsc.cumsum(mi) - mi, trash)
                    plsc.store_scatter(src_v, [pos], sv)
                    if with_scale:
                        cv = scale_v[s]
                    plsc.store_scatter(dst_v, [pos], dv - base)
                    if with_scale:
                        plsc.store_scatter(scale_v, [pos], cv)
                    return cnt + jnp.sum(mi)

                cnt = lax.fori_loop(0, nv, compact, 0)

                for j in range(R // L):
                    src_v[pl.ds(cnt + j * L, L)] = jnp.zeros((L,), jnp.int32)
                    dst_v[pl.ds(cnt + j * L, L)] = jnp.full((L,), C_CHUNK, jnp.int32)
                    if with_scale:
                        scale_v[pl.ds(cnt + j * L, L)] = jnp.zeros((L,), jnp.float32)

                def batch(i, _):
                    bs = i * R
                    pltpu.async_copy(h_hbm.at[src_v.at[pl.ds(bs, R)]], rows_v, sem).wait()
                    if with_scale:
                        def scale_row(r, _):
                            sv16 = plsc.load_gather(scale_v, [jnp.full((L,), bs + r, jnp.int32)])
                            for j in range(D // L):
                                s2 = pl.ds(j * L, L)
                                rows_v[r, s2] = rows_v[r, s2] * sv16
                            return 0
                        lax.fori_loop(0, R, scale_row, 0)
                    for j in range(R // L):
                        ldst2[0, pl.ds(j * L, L)] = dst_v[pl.ds(bs + j * L, L)]
                    pltpu.sync_copy(rows_v, acc.at[ldst2.at[0]], add=True)
                    return 0

                lax.fori_loop(0, (cnt + R - 1) // R, batch, 0)
                plsc.subcore_barrier()

                nbw_full = valid // R
                wtail = valid - nbw_full * R
                for t in range(_ceil(nbw_full, NS)):
                    blk = sid + NS * t
                    if (t + 1) * NS <= nbw_full:
                        pltpu.sync_copy(acc.at[pl.ds(blk * R, R)], rows_v)
                        pltpu.sync_copy(rows_v, out_hbm.at[pl.ds(base + blk * R, R)])
                    else:
                        @pl.when(blk < nbw_full)
                        def _():
                            pltpu.sync_copy(acc.at[pl.ds(blk * R, R)], rows_v)
                            pltpu.sync_copy(rows_v, out_hbm.at[pl.ds(base + blk * R, R)])
                if wtail:
                    @pl.when(sid == NS - 1)
                    def _():
                        pltpu.sync_copy(acc.at[pl.ds(nbw_full * R, wtail)],
                                        rows_v.at[pl.ds(0, wtail)])
                        pltpu.sync_copy(rows_v.at[pl.ds(0, wtail)],
                                        out_hbm.at[pl.ds(base + nbw_full * R, wtail)])
                plsc.subcore_barrier()

    return pl.kernel(
        body,
        out_type=jax.ShapeDtypeStruct((N, D), jnp.float32),
        mesh=_mesh(),
        scratch_types=scratch,
        compiler_params=pltpu.CompilerParams(needs_layout_passes=False),
    )


def _pad_edges(src, dst, n, scale=None):
    e = src.shape[0]
    ep = _rup(e, 2048)
    if ep != e:
        p = ep - e
        src = jnp.concatenate([src, jnp.zeros((p,), jnp.int32)])
        dst = jnp.concatenate([dst, jnp.full((p,), n, jnp.int32)])
        if scale is not None:
            scale = jnp.concatenate([scale, jnp.zeros((p,), jnp.float32)])
    return src, dst, scale


def _sc_rowsum(h, src, dst, n, scale=None):
    src, dst, scale = _pad_edges(src, dst, n, scale)
    kf = _make_rowsum(src.shape[0], n, h.shape[0], scale is not None)
    if scale is None:
        return kf(h, src, dst)
    return kf(h, src, dst, scale)


BN = 1000


def _grid1(n, bn):
    return _ceil(n, bn)


def _tc_mm(x, w):
    n, k = x.shape

    def body(x_ref, w_ref, o_ref):
        o_ref[...] = jnp.dot(x_ref[...], w_ref[...],
                             preferred_element_type=jnp.float32)

    bn = BN if n % BN == 0 else n
    return pl.pallas_call(
        body,
        grid=(n // bn,),
        in_specs=[pl.BlockSpec((bn, k), lambda i: (i, 0)),
                  pl.BlockSpec((k, D), lambda i: (0, 0))],
        out_specs=pl.BlockSpec((bn, D), lambda i: (i, 0)),
        out_shape=jax.ShapeDtypeStruct((n, D), jnp.float32),
    )(x, w)


def _tc_dinv(p, n):
    p0 = p[0, :n].reshape(n, 1)
    p1 = p[1, :n].reshape(n, 1)

    def body(p0_ref, p1_ref, o_ref):
        o_ref[...] = lax.rsqrt(p0_ref[...] + p1_ref[...] + 1.0)

    bn = BN if n % BN == 0 else n
    return pl.pallas_call(
        body,
        grid=(n // bn,),
        in_specs=[pl.BlockSpec((bn, 1), lambda i: (i, 0)),
                  pl.BlockSpec((bn, 1), lambda i: (i, 0))],
        out_specs=pl.BlockSpec((bn, 1), lambda i: (i, 0)),
        out_shape=jax.ShapeDtypeStruct((n, 1), jnp.float32),
    )(p0, p1)


def _tc_mm_scale(x, w, dinv):
    n, k = x.shape

    def body(x_ref, w_ref, d_ref, o_ref):
        h = jnp.dot(x_ref[...], w_ref[...], preferred_element_type=jnp.float32)
        o_ref[...] = h * d_ref[...]

    bn = BN if n % BN == 0 else n
    return pl.pallas_call(
        body,
        grid=(n // bn,),
        in_specs=[pl.BlockSpec((bn, k), lambda i: (i, 0)),
                  pl.BlockSpec((k, D), lambda i: (0, 0)),
                  pl.BlockSpec((bn, 1), lambda i: (i, 0))],
        out_specs=pl.BlockSpec((bn, D), lambda i: (i, 0)),
        out_shape=jax.ShapeDtypeStruct((n, D), jnp.float32),
    )(x, w, dinv)


def _tc_gcn_combine(acc, hs, dinv, b, res=None):
    n = acc.shape[0]
    has_res = res is not None

    def body(*refs):
        if has_res:
            a_ref, h_ref, d_ref, b_ref, r_ref, o_ref = refs
        else:
            a_ref, h_ref, d_ref, b_ref, o_ref = refs
        out = d_ref[...] * (a_ref[...] + h_ref[...]) + b_ref[...]
        if has_res:
            out = out + r_ref[...]
        o_ref[...] = out

    bn = BN if n % BN == 0 else n
    specs = [pl.BlockSpec((bn, D), lambda i: (i, 0)),
             pl.BlockSpec((bn, D), lambda i: (i, 0)),
             pl.BlockSpec((bn, 1), lambda i: (i, 0)),
             pl.BlockSpec((1, D), lambda i: (0, 0))]
    args = [acc, hs, dinv, b.reshape(1, D)]
    if has_res:
        specs.append(pl.BlockSpec((bn, D), lambda i: (i, 0)))
        args.append(res)
    return pl.pallas_call(
        body,
        grid=(n // bn,),
        in_specs=specs,
        out_specs=pl.BlockSpec((bn, D), lambda i: (i, 0)),
        out_shape=jax.ShapeDtypeStruct((n, D), jnp.float32),
    )(*args)


def _tc_gat_head(x, w, a_s, a_d):
    n, k = x.shape
    bn = BN if n % BN == 0 else n
    grid = n // bn

    def body(x_ref, w_ref, as_ref, ad_ref, h_ref, es_ref, ed_ref, esf_ref,
             m_ref, mx_ref):
        i = pl.program_id(0)
        h = jnp.dot(x_ref[...], w_ref[...], preferred_element_type=jnp.float32)
        h_ref[...] = h
        es = jnp.dot(h, as_ref[...], preferred_element_type=jnp.float32)
        ed = jnp.dot(h, ad_ref[...], preferred_element_type=jnp.float32)
        es_ref[...] = es
        ed_ref[...] = ed
        z = es + ed
        esf_ref[...] = jnp.where(z >= 0.0, z, 0.2 * z)
        bmax_s = jnp.max(es)
        bmax_d = jnp.max(ed)

        @pl.when(i == 0)
        def _():
            mx_ref[0] = bmax_s
            mx_ref[1] = bmax_d

        @pl.when(i > 0)
        def _():
            mx_ref[0] = jnp.maximum(mx_ref[0], bmax_s)
            mx_ref[1] = jnp.maximum(mx_ref[1], bmax_d)

        @pl.when(i == grid - 1)
        def _():
            m_ref[...] = jnp.full((1, D), jnp.maximum(mx_ref[0] + mx_ref[1], 0.0),
                                  jnp.float32)

    return pl.pallas_call(
        body,
        grid=(grid,),
        in_specs=[pl.BlockSpec((bn, k), lambda i: (i, 0)),
                  pl.BlockSpec((k, D), lambda i: (0, 0)),
                  pl.BlockSpec((D, 1), lambda i: (0, 0)),
                  pl.BlockSpec((D, 1), lambda i: (0, 0))],
        out_specs=[pl.BlockSpec((bn, D), lambda i: (i, 0)),
                   pl.BlockSpec((bn, 1), lambda i: (i, 0)),
                   pl.BlockSpec((bn, 1), lambda i: (i, 0)),
                   pl.BlockSpec((bn, 1), lambda i: (i, 0)),
                   pl.BlockSpec((1, D), lambda i: (0, 0))],
        out_shape=[jax.ShapeDtypeStruct((n, D), jnp.float32),
                   jax.ShapeDtypeStruct((n, 1), jnp.float32),
                   jax.ShapeDtypeStruct((n, 1), jnp.float32),
                   jax.ShapeDtypeStruct((n, 1), jnp.float32),
                   jax.ShapeDtypeStruct((1, D), jnp.float32)],
        scratch_shapes=[pltpu.SMEM((2,), jnp.float32)],
    )(x, w, a_s.reshape(D, 1), a_d.reshape(D, 1))


def _tc_gat_combine(acc, h, eself, m, denp, b, n):
    p0 = denp[0, :n].reshape(n, 1)
    p1 = denp[1, :n].reshape(n, 1)

    def body(a_ref, h_ref, ef_ref, m_ref, p0_ref, p1_ref, b_ref, o_ref):
        ee_self = jnp.exp(ef_ref[...] - m_ref[0, 0])
        den = p0_ref[...] + p1_ref[...] + ee_self
        o_ref[...] = (a_ref[...] + ee_self * h_ref[...]) / (den + 1e-16) + b_ref[...]

    bn = BN if n % BN == 0 else n
    return pl.pallas_call(
        body,
        grid=(n // bn,),
        in_specs=[pl.BlockSpec((bn, D), lambda i: (i, 0)),
                  pl.BlockSpec((bn, D), lambda i: (i, 0)),
                  pl.BlockSpec((bn, 1), lambda i: (i, 0)),
                  pl.BlockSpec((1, D), lambda i: (0, 0)),
                  pl.BlockSpec((bn, 1), lambda i: (i, 0)),
                  pl.BlockSpec((bn, 1), lambda i: (i, 0)),
                  pl.BlockSpec((1, D), lambda i: (0, 0))],
        out_specs=pl.BlockSpec((bn, D), lambda i: (i, 0)),
        out_shape=jax.ShapeDtypeStruct((n, D), jnp.float32),
    )(acc, h, eself, m, p0, p1, b.reshape(1, D))


def _tc_add3(a, b, c):
    n = a.shape[0]

    def body(a_ref, b_ref, c_ref, o_ref):
        o_ref[...] = a_ref[...] + b_ref[...] + c_ref[...]

    bn = BN if n % BN == 0 else n
    return pl.pallas_call(
        body,
        grid=(n // bn,),
        in_specs=[pl.BlockSpec((bn, D), lambda i: (i, 0))] * 3,
        out_specs=pl.BlockSpec((bn, D), lambda i: (i, 0)),
        out_shape=jax.ShapeDtypeStruct((n, D), jnp.float32),
    )(a, b, c)


def _tc_l2norm(x):
    n = x.shape[0]

    def body(x_ref, o_ref):
        v = x_ref[...]
        nrm = jnp.sqrt(jnp.sum(v * v, axis=1, keepdims=True))
        o_ref[...] = v / jnp.maximum(nrm, 1e-12)

    bn = BN if n % BN == 0 else n
    return pl.pallas_call(
        body,
        grid=(n // bn,),
        in_specs=[pl.BlockSpec((bn, D), lambda i: (i, 0))],
        out_specs=pl.BlockSpec((bn, D), lambda i: (i, 0)),
        out_shape=jax.ShapeDtypeStruct((n, D), jnp.float32),
    )(x)


USE_SC_HIST = True
USE_SC_EDGE = True
USE_SC_ROWSUM = True


def _deg_dinv(dst, n):
    if USE_SC_HIST:
        degp, _ = _sc_seghist(dst, n)
        return _tc_dinv(degp, n)
    deg = jax.ops.segment_sum(jnp.ones_like(dst, jnp.float32), dst, num_segments=n)
    return lax.rsqrt(deg + 1.0).reshape(n, 1)


def _gcn_layer(x, src, dst, dinv, W, b, n, res=None):
    hs = _tc_mm_scale(x, W, dinv)
    if USE_SC_ROWSUM:
        acc = _sc_rowsum(hs, src, dst, n)
    else:
        acc = jax.ops.segment_sum(hs[src], dst, num_segments=n)
    return _tc_gcn_combine(acc, hs, dinv, b, res)


def _gat_layer(x, src, dst, W, a_s, a_d, b, n):
    E = src.shape[0]
    h, es, ed, eself, m = _tc_gat_head(x, W, a_s, a_d)
    if USE_SC_EDGE:
        p1 = _make_edge_pass(E, n, 1)(es.reshape(n), src)
        ee = _make_edge_pass(E, n, 2)(ed.reshape(n), dst, p1, m.reshape(D))
    else:
        z = es.reshape(n)[src] + ed.reshape(n)[dst]
        ee = jnp.exp(jnp.where(z >= 0.0, z, 0.2 * z) - m.reshape(D)[0])
    if USE_SC_HIST:
        denp, _ = _sc_seghist(dst, n, ee)
    else:
        den = jax.ops.segment_sum(ee, dst, num_segments=n)
        denp = jnp.stack([den, jnp.zeros_like(den)])
    if USE_SC_ROWSUM:
        acc = _sc_rowsum(h, src, dst, n, ee)
    else:
        acc = jax.ops.segment_sum(h[src] * ee[:, None], dst, num_segments=n)
    return _tc_gat_combine(acc, h, eself, m, denp, b, n)


def kernel(ent_seed_sr, ent_seed_tg, attribute_triples_sr, attribute_triples_tg, edges_sr, edges_tg, ev_edges_sr, vv_edges_sr, ev_edges_tg, vv_edges_tg, val_feats, att_feats, ent_feats_sr, ent_feats_tg, W_ve, v_gcn1_W, v_gcn1_b, v_gcn2_W, v_gcn2_b, gat1_W, gat1_as, gat1_ad, gat1_b, gatr_W, gatr_as, gatr_ad, gatr_b, e_gcn1_W, e_gcn1_b, e_gcn2_W, e_gcn2_b):
    att_pad = jnp.concatenate(
        [att_feats, jnp.zeros((11, D), jnp.float32)], axis=0)
    PA = _tc_mm(att_pad, W_ve[:D])
    PV = _tc_mm(val_feats, W_ve[D:])

    def value_encoder(triples, ent_edges, val_edges, ent_feats):
        vf = _sc_gather2(PA, triples[:, 2], PV, triples[:, 1])
        nodes = jnp.concatenate([ent_feats, vf], axis=0)
        n = nodes.shape[0]
        vsrc, vdst = val_edges[:, 0], val_edges[:, 1]
        dinv = _deg_dinv(vdst, n)
        nodes = _gcn_layer(nodes, vsrc, vdst, dinv, v_gcn1_W, v_gcn1_b, n)
        nodes = _gcn_layer(nodes, vsrc, vdst, dinv, v_gcn2_W, v_gcn2_b, n)
        esrc, edst = ent_edges[:, 0], ent_edges[:, 1]
        g1 = _gat_layer(nodes, esrc, edst, gat1_W, gat1_as, gat1_ad, gat1_b, n)
        g2 = _gat_layer(nodes, esrc, edst, gatr_W, gatr_as, gatr_ad, gatr_b, n)
        num_ent = ent_feats.shape[0]
        return _tc_add3(g1[:num_ent], g2[:num_ent], ent_feats)

    def entity_encoder(x, edges):
        n = x.shape[0]
        src, dst = edges[:, 0], edges[:, 1]
        dinv = _deg_dinv(dst, n)
        h = _gcn_layer(x, src, dst, dinv, e_gcn1_W, e_gcn1_b, n, res=x)
        h = _gcn_layer(h, src, dst, dinv, e_gcn2_W, e_gcn2_b, n, res=x)
        return h

    efs = value_encoder(attribute_triples_sr, ev_edges_sr, vv_edges_sr, ent_feats_sr)
    eft = value_encoder(attribute_triples_tg, ev_edges_tg, vv_edges_tg, ent_feats_tg)
    efs = entity_encoder(efs, edges_sr)
    eft = entity_encoder(eft, edges_tg)
    efs = _tc_l2norm(efs)
    eft = _tc_l2norm(eft)
    return (_sc_gather(efs, ent_seed_sr), _sc_gather(eft, ent_seed_tg), efs, eft)


# speedup vs baseline: 4.9442x; 1.0489x over previous
import functools

import jax
import jax.numpy as jnp
from jax import lax
from jax.experimental import pallas as pl
from jax.experimental.pallas import tpu as pltpu
from jax.experimental.pallas import tpu_sc as plsc

NC, NS, L = 2, 16, 16
NW = NC * NS
D = 128
R = 128


def _mesh():
    return plsc.VectorSubcoreMesh(core_axis_name="c", subcore_axis_name="s")


def _wid():
    return lax.axis_index("s") * NC + lax.axis_index("c")


def _ceil(a, b):
    return (a + b - 1) // b


def _rup(a, b):
    return _ceil(a, b) * b


@functools.lru_cache(maxsize=None)
def _make_gather(n_idx, na, nb):
    two = nb is not None
    nb_full = n_idx // R
    tail = n_idx - nb_full * R
    T = _ceil(nb_full, NW)

    scratch = [pltpu.VMEM((R,), jnp.int32), pltpu.VMEM((R, D), jnp.float32)]
    if two:
        scratch += [pltpu.VMEM((R,), jnp.int32), pltpu.VMEM((R, D), jnp.float32)]
    scratch += [pltpu.SemaphoreType.DMA]

    def body(*refs):
        if two:
            a_hbm, ia_hbm, b_hbm, ib_hbm, out_hbm, ia_v, rows_a, ib_v, rows_b, sem = refs
        else:
            a_hbm, ia_hbm, out_hbm, ia_v, rows_a, sem = refs
        w = _wid()

        def zfill(ref, nrows):
            if nrows < R:
                for o in range((nrows // L) * L, R, L):
                    ref[pl.ds(o, L)] = jnp.zeros((L,), jnp.int32)

        def process(start, nrows):
            zfill(ia_v, nrows)
            pltpu.sync_copy(ia_hbm.at[pl.ds(start, nrows)], ia_v.at[pl.ds(0, nrows)])
            pltpu.async_copy(a_hbm.at[ia_v], rows_a, sem).wait()
            if two:
                zfill(ib_v, nrows)
                pltpu.sync_copy(ib_hbm.at[pl.ds(start, nrows)], ib_v.at[pl.ds(0, nrows)])
                pltpu.async_copy(b_hbm.at[ib_v], rows_b, sem).wait()

                def add_row(r, _):
                    for j in range(D // L):
                        s = pl.ds(j * L, L)
                        rows_a[r, s] = rows_a[r, s] + rows_b[r, s]
                    return 0

                lax.fori_loop(0, nrows, add_row, 0)
            pltpu.sync_copy(rows_a.at[pl.ds(0, nrows)], out_hbm.at[pl.ds(start, nrows)])

        for t in range(T):
            b = w + NW * t
            if (t + 1) * NW <= nb_full:
                process(b * R, R)
            else:
                @pl.when(b < nb_full)
                def _():
                    process(b * R, R)
        if tail:
            @pl.when(w == NW - 1)
            def _():
                process(nb_full * R, tail)

    return pl.kernel(
        body,
        out_type=jax.ShapeDtypeStruct((n_idx, D), jnp.float32),
        mesh=_mesh(),
        scratch_types=scratch,
        compiler_params=pltpu.CompilerParams(needs_layout_passes=False),
    )


def _sc_gather(a, ia):
    return _make_gather(ia.shape[0], a.shape[0], None)(a, ia)


def _sc_gather2(a, ia, b, ib):
    return _make_gather(ia.shape[0], a.shape[0], b.shape[0])(a, ia, b, ib)


@functools.lru_cache(maxsize=None)
def _make_seghist(E, N, with_vals):
    NA = _rup(N + 1, 2048)
    stripe = NA // NS
    nb_full = E // R
    tail = E - nb_full * R
    T = _ceil(nb_full, NW)

    scratch = [
        pltpu.VMEM((R,), jnp.int32),
        pltpu.VMEM((R,), jnp.float32),
        pltpu.VMEM((R,), jnp.float32),
        pltpu.VMEM_SHARED((NA,), jnp.float32),
        pltpu.SemaphoreType.DMA,
    ]

    def body(*refs):
        if with_vals:
            dst_hbm, vals_hbm, out_hbm, ldst_v, vals_v, zb, acc, sem = refs
        else:
            dst_hbm, out_hbm, ldst_v, vals_v, zb, acc, sem = refs
        cid = lax.axis_index("c")
        sid = lax.axis_index("s")
        w = sid * NC + cid

        for j in range(R // L):
            zb[pl.ds(j * L, L)] = jnp.zeros((L,), jnp.float32)
        if not with_vals:
            for j in range(R // L):
                vals_v[pl.ds(j * L, L)] = jnp.ones((L,), jnp.float32)

        def zrow(t, _):
            pltpu.sync_copy(zb, acc.at[pl.ds(sid * stripe + t * R, R)])
            return 0
        lax.fori_loop(0, stripe // R, zrow, 0)
        plsc.subcore_barrier()

        def process(start, nrows):
            if nrows < R:
                for o in range((nrows // L) * L, R, L):
                    ldst_v[pl.ds(o, L)] = jnp.full((L,), N, jnp.int32)
            pltpu.sync_copy(dst_hbm.at[pl.ds(start, nrows)], ldst_v.at[pl.ds(0, nrows)])
            if with_vals:
                pltpu.sync_copy(vals_hbm.at[pl.ds(start, nrows)], vals_v.at[pl.ds(0, nrows)])
            pltpu.sync_copy(vals_v, acc.at[ldst_v], add=True)

        for t in range(T):
            b = w + NW * t
            if (t + 1) * NW <= nb_full:
                process(b * R, R)
            else:
                @pl.when(b < nb_full)
                def _():
                    process(b * R, R)
        if tail:
            @pl.when(w == NW - 1)
            def _():
                process(nb_full * R, tail)

        plsc.subcore_barrier()
        def wrow(t, _):
            o = sid * stripe + t * R
            pltpu.sync_copy(acc.at[pl.ds(o, R)], zb)
            pltpu.sync_copy(zb, out_hbm.at[cid, pl.ds(o, R)])
            return 0
        lax.fori_loop(0, stripe // R, wrow, 0)

    return pl.kernel(
        body,
        out_type=jax.ShapeDtypeStruct((2, NA), jnp.float32),
        mesh=_mesh(),
        scratch_types=scratch,
        compiler_params=pltpu.CompilerParams(needs_layout_passes=False),
    ), NA


def _sc_seghist(dst, n, vals=None):
    kf, na = _make_seghist(dst.shape[0], n, vals is not None)
    if vals is None:
        return kf(dst), na
    return kf(dst, vals), na


@functools.lru_cache(maxsize=None)
def _make_edge_pass(E, N, phase):
    nb_full = E // R
    tail = E - nb_full * R
    T = _ceil(nb_full, NW)

    scratch = [
        pltpu.VMEM((N,), jnp.float32),
        pltpu.VMEM((R,), jnp.int32),
        pltpu.VMEM((R,), jnp.float32),
        pltpu.SemaphoreType.DMA,
    ]
    if phase == 2:
        scratch.insert(2, pltpu.VMEM((R,), jnp.float32))
        scratch.insert(3, pltpu.VMEM((16,), jnp.float32))

    def body(*refs):
        if phase == 1:
            tab_hbm, idx_hbm, out_hbm, tab_v, idx_v, out_v, sem = refs
        else:
            tab_hbm, idx_hbm, p1_hbm, m_hbm, out_hbm, tab_v, idx_v, p1_v, m_v, out_v, sem = refs
        w = _wid()
        pltpu.sync_copy(tab_hbm, tab_v)
        if phase == 2:
            pltpu.sync_copy(m_hbm.at[pl.ds(0, L)], m_v)

        def process(start, nrows):
            if nrows < R:
                for o in range((nrows // L) * L, R, L):
                    idx_v[pl.ds(o, L)] = jnp.zeros((L,), jnp.int32)
            pltpu.sync_copy(idx_hbm.at[pl.ds(start, nrows)], idx_v.at[pl.ds(0, nrows)])
            if phase == 2:
                pltpu.sync_copy(p1_hbm.at[pl.ds(start, nrows)], p1_v.at[pl.ds(0, nrows)])
                mv = m_v[pl.ds(0, L)]
            for j in range(R // L):
                s = pl.ds(j * L, L)
                g = plsc.load_gather(tab_v, [idx_v[s]])
                if phase == 1:
                    out_v[s] = g
                else:
                    x = p1_v[s] + g
                    x = jnp.where(x >= 0.0, x, 0.2 * x)
                    out_v[s] = jnp.exp(x - mv)
            pltpu.sync_copy(out_v.at[pl.ds(0, nrows)], out_hbm.at[pl.ds(start, nrows)])

        for t in range(T):
            b = w + NW * t
            if (t + 1) * NW <= nb_full:
                process(b * R, R)
            else:
                @pl.when(b < nb_full)
                def _():
                    process(b * R, R)
        if tail:
            @pl.when(w == NW - 1)
            def _():
                process(nb_full * R, tail)

    return pl.kernel(
        body,
        out_type=jax.ShapeDtypeStruct((E,), jnp.float32),
        mesh=_mesh(),
        scratch_types=scratch,
        compiler_params=pltpu.CompilerParams(needs_layout_passes=False),
    )


C_CHUNK = 4080
CA = C_CHUNK + 16


@functools.lru_cache(maxsize=None)
def _make_rowsum(E, N, Ns, with_scale):
    assert E % 2048 == 0
    E16 = E // NS
    nchunks = _ceil(N, C_CHUNK)
    nv = E16 // L

    scratch = [
        pltpu.VMEM((E16 + R + L,), jnp.int32),
        pltpu.VMEM((E16 + R + L,), jnp.int32),
        pltpu.VMEM((2, R, D), jnp.float32),
        pltpu.VMEM((R, D), jnp.float32),
        pltpu.VMEM((1, R), jnp.int32),
        pltpu.VMEM_SHARED((CA, D), jnp.float32),
        pltpu.SemaphoreType.DMA,
    ]
    if with_scale:
        scratch.insert(2, pltpu.VMEM((E16 + R + L,), jnp.float32))

    def body(*refs):
        if with_scale:
            (h_hbm, src_hbm, dst_hbm, scale_hbm, out_hbm,
             src_v, dst_v, scale_v, rows_v, zb, ldst2, acc, sem) = refs
        else:
            (h_hbm, src_hbm, dst_hbm, out_hbm,
             src_v, dst_v, rows_v, zb, ldst2, acc, sem) = refs
        cid = lax.axis_index("c")
        sid = lax.axis_index("s")

        def zrow(r, _):
            for j in range(D // L):
                zb[r, pl.ds(j * L, L)] = jnp.zeros((L,), jnp.float32)
            return 0
        lax.fori_loop(0, R, zrow, 0)

        pltpu.sync_copy(src_hbm.at[pl.ds(sid * E16, E16)], src_v.at[pl.ds(0, E16)])
        pltpu.sync_copy(dst_hbm.at[pl.ds(sid * E16, E16)], dst_v.at[pl.ds(0, E16)])
        if with_scale:
            pltpu.sync_copy(scale_hbm.at[pl.ds(sid * E16, E16)], scale_v.at[pl.ds(0, E16)])

        for q in range(nchunks):
            base = q * C_CHUNK
            valid = min(C_CHUNK, N - base)

            @pl.when(cid == (q % 2))
            def _():
                for t in range(CA // NS // R):
                    pltpu.sync_copy(zb, acc.at[pl.ds(sid * (CA // NS) + t * R, R)])
                plsc.subcore_barrier()

                if q == 0:
                    pass
                else:
                    pltpu.sync_copy(src_hbm.at[pl.ds(sid * E16, E16)],
                                    src_v.at[pl.ds(0, E16)])
                    pltpu.sync_copy(dst_hbm.at[pl.ds(sid * E16, E16)],
                                    dst_v.at[pl.ds(0, E16)])
                    if with_scale:
                        pltpu.sync_copy(scale_hbm.at[pl.ds(sid * E16, E16)],
                                        scale_v.at[pl.ds(0, E16)])

                trash = E16 + R

                def compact(t, cnt):
                    s = pl.ds(t * L, L)
                    dv = dst_v[s]
                    sv = src_v[s]
                    msk = (dv >= base) & (dv < base + C_CHUNK)
                    mi = msk.astype(jnp.int32)
                    pos = jnp.where(msk, cnt + plsc.cumsum(mi) - mi, trash)
                    plsc.store_scatter(src_v, [pos], sv)
                    if with_scale:
                        cv = scale_v[s]
                    plsc.store_scatter(dst_v, [pos], dv - base)
                    if with_scale:
                        plsc.store_scatter(scale_v, [pos], cv)
                    return cnt + jnp.sum(mi)

                cnt = lax.fori_loop(0, nv, compact, 0)

                for j in range(R // L):
                    src_v[pl.ds(cnt + j * L, L)] = jnp.zeros((L,), jnp.int32)
                    dst_v[pl.ds(cnt + j * L, L)] = jnp.full((L,), C_CHUNK, jnp.int32)
                    if with_scale:
                        scale_v[pl.ds(cnt + j * L, L)] = jnp.zeros((L,), jnp.float32)

                nb = (cnt + R - 1) // R

                def fire(i):
                    pltpu.async_copy(h_hbm.at[src_v.at[pl.ds(i * R, R)]],
                                     rows_v.at[i % 2], sem)

                @pl.when(nb > 0)
                def _():
                    fire(0)

                def batch(i, _):
                    bs = i * R
                    rv = rows_v.at[i % 2]
                    pltpu.make_async_copy(h_hbm.at[pl.ds(0, R)], rv, sem).wait()

                    @pl.when(i + 1 < nb)
                    def _():
                        fire(i + 1)

                    if with_scale:
                        def scale_row(r, _):
                            sv16 = plsc.load_gather(scale_v, [jnp.full((L,), bs + r, jnp.int32)])
                            for j in range(D // L):
                                s2 = pl.ds(j * L, L)
                                rv[r, s2] = rv[r, s2] * sv16
                            return 0
                        lax.fori_loop(0, R, scale_row, 0)
                    for j in range(R // L):
                        ldst2[0, pl.ds(j * L, L)] = dst_v[pl.ds(bs + j * L, L)]
                    pltpu.sync_copy(rv, acc.at[ldst2.at[0]], add=True)
                    return 0

                lax.fori_loop(0, nb, batch, 0)
                plsc.subcore_barrier()

                nbw_full = valid // R
                wtail = valid - nbw_full * R
                stage = rows_v.at[0]
                for t in range(_ceil(nbw_full, NS)):
                    blk = sid + NS * t
                    if (t + 1) * NS <= nbw_full:
                        pltpu.sync_copy(acc.at[pl.ds(blk * R, R)], stage)
                        pltpu.sync_copy(stage, out_hbm.at[pl.ds(base + blk * R, R)])
                    else:
                        @pl.when(blk < nbw_full)
                        def _():
                            pltpu.sync_copy(acc.at[pl.ds(blk * R, R)], stage)
                            pltpu.sync_copy(stage, out_hbm.at[pl.ds(base + blk * R, R)])
                if wtail:
                    @pl.when(sid == NS - 1)
                    def _():
                        pltpu.sync_copy(acc.at[pl.ds(nbw_full * R, wtail)],
                                        stage.at[pl.ds(0, wtail)])
                        pltpu.sync_copy(stage.at[pl.ds(0, wtail)],
                                        out_hbm.at[pl.ds(base + nbw_full * R, wtail)])
                plsc.subcore_barrier()

    return pl.kernel(
        body,
        out_type=jax.ShapeDtypeStruct((N, D), jnp.float32),
        mesh=_mesh(),
        scratch_types=scratch,
        compiler_params=pltpu.CompilerParams(needs_layout_passes=False),
    )


def _pad_edges(src, dst, n, scale=None):
    e = src.shape[0]
    ep = _rup(e, 2048)
    if ep != e:
        p = ep - e
        src = jnp.concatenate([src, jnp.zeros((p,), jnp.int32)])
        dst = jnp.concatenate([dst, jnp.full((p,), n, jnp.int32)])
        if scale is not None:
            scale = jnp.concatenate([scale, jnp.zeros((p,), jnp.float32)])
    return src, dst, scale


def _sc_rowsum(h, src, dst, n, scale=None):
    src, dst, scale = _pad_edges(src, dst, n, scale)
    kf = _make_rowsum(src.shape[0], n, h.shape[0], scale is not None)
    if scale is None:
        return kf(h, src, dst)
    return kf(h, src, dst, scale)


BN = 1000


def _grid1(n, bn):
    return _ceil(n, bn)


def _tc_mm(x, w):
    n, k = x.shape

    def body(x_ref, w_ref, o_ref):
        o_ref[...] = jnp.dot(x_ref[...], w_ref[...],
                             preferred_element_type=jnp.float32)

    bn = BN if n % BN == 0 else n
    return pl.pallas_call(
        body,
        grid=(n // bn,),
        in_specs=[pl.BlockSpec((bn, k), lambda i: (i, 0)),
                  pl.BlockSpec((k, D), lambda i: (0, 0))],
        out_specs=pl.BlockSpec((bn, D), lambda i: (i, 0)),
        out_shape=jax.ShapeDtypeStruct((n, D), jnp.float32),
    )(x, w)


def _tc_dinv(p, n):
    p0 = p[0, :n].reshape(n, 1)
    p1 = p[1, :n].reshape(n, 1)

    def body(p0_ref, p1_ref, o_ref):
        o_ref[...] = lax.rsqrt(p0_ref[...] + p1_ref[...] + 1.0)

    bn = BN if n % BN == 0 else n
    return pl.pallas_call(
        body,
        grid=(n // bn,),
        in_specs=[pl.BlockSpec((bn, 1), lambda i: (i, 0)),
                  pl.BlockSpec((bn, 1), lambda i: (i, 0))],
        out_specs=pl.BlockSpec((bn, 1), lambda i: (i, 0)),
        out_shape=jax.ShapeDtypeStruct((n, 1), jnp.float32),
    )(p0, p1)


def _tc_mm_scale(x, w, dinv):
    n, k = x.shape

    def body(x_ref, w_ref, d_ref, o_ref):
        h = jnp.dot(x_ref[...], w_ref[...], preferred_element_type=jnp.float32)
        o_ref[...] = h * d_ref[...]

    bn = BN if n % BN == 0 else n
    return pl.pallas_call(
        body,
        grid=(n // bn,),
        in_specs=[pl.BlockSpec((bn, k), lambda i: (i, 0)),
                  pl.BlockSpec((k, D), lambda i: (0, 0)),
                  pl.BlockSpec((bn, 1), lambda i: (i, 0))],
        out_specs=pl.BlockSpec((bn, D), lambda i: (i, 0)),
        out_shape=jax.ShapeDtypeStruct((n, D), jnp.float32),
    )(x, w, dinv)


def _tc_gcn_combine(acc, hs, dinv, b, res=None):
    n = acc.shape[0]
    has_res = res is not None

    def body(*refs):
        if has_res:
            a_ref, h_ref, d_ref, b_ref, r_ref, o_ref = refs
        else:
            a_ref, h_ref, d_ref, b_ref, o_ref = refs
        out = d_ref[...] * (a_ref[...] + h_ref[...]) + b_ref[...]
        if has_res:
            out = out + r_ref[...]
        o_ref[...] = out

    bn = BN if n % BN == 0 else n
    specs = [pl.BlockSpec((bn, D), lambda i: (i, 0)),
             pl.BlockSpec((bn, D), lambda i: (i, 0)),
             pl.BlockSpec((bn, 1), lambda i: (i, 0)),
             pl.BlockSpec((1, D), lambda i: (0, 0))]
    args = [acc, hs, dinv, b.reshape(1, D)]
    if has_res:
        specs.append(pl.BlockSpec((bn, D), lambda i: (i, 0)))
        args.append(res)
    return pl.pallas_call(
        body,
        grid=(n // bn,),
        in_specs=specs,
        out_specs=pl.BlockSpec((bn, D), lambda i: (i, 0)),
        out_shape=jax.ShapeDtypeStruct((n, D), jnp.float32),
    )(*args)


def _tc_gat_head(x, w, a_s, a_d):
    n, k = x.shape
    bn = BN if n % BN == 0 else n
    grid = n // bn

    def body(x_ref, w_ref, as_ref, ad_ref, h_ref, es_ref, ed_ref, esf_ref,
             m_ref, mx_ref):
        i = pl.program_id(0)
        h = jnp.dot(x_ref[...], w_ref[...], preferred_element_type=jnp.float32)
        h_ref[...] = h
        es = jnp.dot(h, as_ref[...], preferred_element_type=jnp.float32)
        ed = jnp.dot(h, ad_ref[...], preferred_element_type=jnp.float32)
        es_ref[...] = es
        ed_ref[...] = ed
        z = es + ed
        esf_ref[...] = jnp.where(z >= 0.0, z, 0.2 * z)
        bmax_s = jnp.max(es)
        bmax_d = jnp.max(ed)

        @pl.when(i == 0)
        def _():
            mx_ref[0] = bmax_s
            mx_ref[1] = bmax_d

        @pl.when(i > 0)
        def _():
            mx_ref[0] = jnp.maximum(mx_ref[0], bmax_s)
            mx_ref[1] = jnp.maximum(mx_ref[1], bmax_d)

        @pl.when(i == grid - 1)
        def _():
            m_ref[...] = jnp.full((1, D), jnp.maximum(mx_ref[0] + mx_ref[1], 0.0),
                                  jnp.float32)

    return pl.pallas_call(
        body,
        grid=(grid,),
        in_specs=[pl.BlockSpec((bn, k), lambda i: (i, 0)),
                  pl.BlockSpec((k, D), lambda i: (0, 0)),
                  pl.BlockSpec((D, 1), lambda i: (0, 0)),
                  pl.BlockSpec((D, 1), lambda i: (0, 0))],
        out_specs=[pl.BlockSpec((bn, D), lambda i: (i, 0)),
                   pl.BlockSpec((bn, 1), lambda i: (i, 0)),
                   pl.BlockSpec((bn, 1), lambda i: (i, 0)),
                   pl.BlockSpec((bn, 1), lambda i: (i, 0)),
                   pl.BlockSpec((1, D), lambda i: (0, 0))],
        out_shape=[jax.ShapeDtypeStruct((n, D), jnp.float32),
                   jax.ShapeDtypeStruct((n, 1), jnp.float32),
                   jax.ShapeDtypeStruct((n, 1), jnp.float32),
                   jax.ShapeDtypeStruct((n, 1), jnp.float32),
                   jax.ShapeDtypeStruct((1, D), jnp.float32)],
        scratch_shapes=[pltpu.SMEM((2,), jnp.float32)],
    )(x, w, a_s.reshape(D, 1), a_d.reshape(D, 1))


def _tc_gat_combine(acc, h, eself, m, denp, b, n):
    p0 = denp[0, :n].reshape(n, 1)
    p1 = denp[1, :n].reshape(n, 1)

    def body(a_ref, h_ref, ef_ref, m_ref, p0_ref, p1_ref, b_ref, o_ref):
        ee_self = jnp.exp(ef_ref[...] - m_ref[0, 0])
        den = p0_ref[...] + p1_ref[...] + ee_self
        o_ref[...] = (a_ref[...] + ee_self * h_ref[...]) / (den + 1e-16) + b_ref[...]

    bn = BN if n % BN == 0 else n
    return pl.pallas_call(
        body,
        grid=(n // bn,),
        in_specs=[pl.BlockSpec((bn, D), lambda i: (i, 0)),
                  pl.BlockSpec((bn, D), lambda i: (i, 0)),
                  pl.BlockSpec((bn, 1), lambda i: (i, 0)),
                  pl.BlockSpec((1, D), lambda i: (0, 0)),
                  pl.BlockSpec((bn, 1), lambda i: (i, 0)),
                  pl.BlockSpec((bn, 1), lambda i: (i, 0)),
                  pl.BlockSpec((1, D), lambda i: (0, 0))],
        out_specs=pl.BlockSpec((bn, D), lambda i: (i, 0)),
        out_shape=jax.ShapeDtypeStruct((n, D), jnp.float32),
    )(acc, h, eself, m, p0, p1, b.reshape(1, D))


def _tc_add3(a, b, c):
    n = a.shape[0]

    def body(a_ref, b_ref, c_ref, o_ref):
        o_ref[...] = a_ref[...] + b_ref[...] + c_ref[...]

    bn = BN if n % BN == 0 else n
    return pl.pallas_call(
        body,
        grid=(n // bn,),
        in_specs=[pl.BlockSpec((bn, D), lambda i: (i, 0))] * 3,
        out_specs=pl.BlockSpec((bn, D), lambda i: (i, 0)),
        out_shape=jax.ShapeDtypeStruct((n, D), jnp.float32),
    )(a, b, c)


def _tc_l2norm(x):
    n = x.shape[0]

    def body(x_ref, o_ref):
        v = x_ref[...]
        nrm = jnp.sqrt(jnp.sum(v * v, axis=1, keepdims=True))
        o_ref[...] = v / jnp.maximum(nrm, 1e-12)

    bn = BN if n % BN == 0 else n
    return pl.pallas_call(
        body,
        grid=(n // bn,),
        in_specs=[pl.BlockSpec((bn, D), lambda i: (i, 0))],
        out_specs=pl.BlockSpec((bn, D), lambda i: (i, 0)),
        out_shape=jax.ShapeDtypeStruct((n, D), jnp.float32),
    )(x)


USE_SC_HIST = True
USE_SC_EDGE = True
USE_SC_ROWSUM = True


def _deg_dinv(dst, n):
    if USE_SC_HIST:
        degp, _ = _sc_seghist(dst, n)
        return _tc_dinv(degp, n)
    deg = jax.ops.segment_sum(jnp.ones_like(dst, jnp.float32), dst, num_segments=n)
    return lax.rsqrt(deg + 1.0).reshape(n, 1)


def _gcn_layer(x, src, dst, dinv, W, b, n, res=None):
    hs = _tc_mm_scale(x, W, dinv)
    if USE_SC_ROWSUM:
        acc = _sc_rowsum(hs, src, dst, n)
    else:
        acc = jax.ops.segment_sum(hs[src], dst, num_segments=n)
    return _tc_gcn_combine(acc, hs, dinv, b, res)


def _gat_layer(x, src, dst, W, a_s, a_d, b, n):
    E = src.shape[0]
    h, es, ed, eself, m = _tc_gat_head(x, W, a_s, a_d)
    if USE_SC_EDGE:
        p1 = _make_edge_pass(E, n, 1)(es.reshape(n), src)
        ee = _make_edge_pass(E, n, 2)(ed.reshape(n), dst, p1, m.reshape(D))
    else:
        z = es.reshape(n)[src] + ed.reshape(n)[dst]
        ee = jnp.exp(jnp.where(z >= 0.0, z, 0.2 * z) - m.reshape(D)[0])
    if USE_SC_HIST:
        denp, _ = _sc_seghist(dst, n, ee)
    else:
        den = jax.ops.segment_sum(ee, dst, num_segments=n)
        denp = jnp.stack([den, jnp.zeros_like(den)])
    if USE_SC_ROWSUM:
        acc = _sc_rowsum(h, src, dst, n, ee)
    else:
        acc = jax.ops.segment_sum(h[src] * ee[:, None], dst, num_segments=n)
    return _tc_gat_combine(acc, h, eself, m, denp, b, n)


def kernel(ent_seed_sr, ent_seed_tg, attribute_triples_sr, attribute_triples_tg, edges_sr, edges_tg, ev_edges_sr, vv_edges_sr, ev_edges_tg, vv_edges_tg, val_feats, att_feats, ent_feats_sr, ent_feats_tg, W_ve, v_gcn1_W, v_gcn1_b, v_gcn2_W, v_gcn2_b, gat1_W, gat1_as, gat1_ad, gat1_b, gatr_W, gatr_as, gatr_ad, gatr_b, e_gcn1_W, e_gcn1_b, e_gcn2_W, e_gcn2_b):
    att_pad = jnp.concatenate(
        [att_feats, jnp.zeros((11, D), jnp.float32)], axis=0)
    PA = _tc_mm(att_pad, W_ve[:D])
    PV = _tc_mm(val_feats, W_ve[D:])

    def value_encoder(triples, ent_edges, val_edges, ent_feats):
        vf = _sc_gather2(PA, triples[:, 2], PV, triples[:, 1])
        nodes = jnp.concatenate([ent_feats, vf], axis=0)
        n = nodes.shape[0]
        vsrc, vdst = val_edges[:, 0], val_edges[:, 1]
        dinv = _deg_dinv(vdst, n)
        nodes = _gcn_layer(nodes, vsrc, vdst, dinv, v_gcn1_W, v_gcn1_b, n)
        nodes = _gcn_layer(nodes, vsrc, vdst, dinv, v_gcn2_W, v_gcn2_b, n)
        esrc, edst = ent_edges[:, 0], ent_edges[:, 1]
        g1 = _gat_layer(nodes, esrc, edst, gat1_W, gat1_as, gat1_ad, gat1_b, n)
        g2 = _gat_layer(nodes, esrc, edst, gatr_W, gatr_as, gatr_ad, gatr_b, n)
        num_ent = ent_feats.shape[0]
        return _tc_add3(g1[:num_ent], g2[:num_ent], ent_feats)

    def entity_encoder(x, edges):
        n = x.shape[0]
        src, dst = edges[:, 0], edges[:, 1]
        dinv = _deg_dinv(dst, n)
        h = _gcn_layer(x, src, dst, dinv, e_gcn1_W, e_gcn1_b, n, res=x)
        h = _gcn_layer(h, src, dst, dinv, e_gcn2_W, e_gcn2_b, n, res=x)
        return h

    efs = value_encoder(attribute_triples_sr, ev_edges_sr, vv_edges_sr, ent_feats_sr)
    eft = value_encoder(attribute_triples_tg, ev_edges_tg, vv_edges_tg, ent_feats_tg)
    efs = entity_encoder(efs, edges_sr)
    eft = entity_encoder(eft, edges_tg)
    efs = _tc_l2norm(efs)
    eft = _tc_l2norm(eft)
    return (_sc_gather(efs, ent_seed_sr), _sc_gather(eft, ent_seed_tg), efs, eft)
